# baseline (speedup 1.0000x reference)
import functools

import jax
import jax.numpy as jnp
from jax import lax
from jax.experimental import pallas as pl
from jax.experimental.pallas import tpu as pltpu
from jax.experimental.pallas import tpu_sc as plsc

_DIM2 = 32
_BATCH = 16384
_FIELDS = 26
_B = _BATCH * _FIELDS

_info = plsc.get_sparse_core_info()
_NC = _info.num_cores
_NS = _info.num_subcores
_NW = _NC * _NS
_B_PER_W = _B // _NW
_BW = _BATCH // _NW

_CH = 104
_NCH = _B_PER_W // _CH
_SUBB = 32
_SUB = _SUBB * _FIELDS
_CPS = _SUB // _CH
_NSUB = _BW // _SUBB
_NP = _NSUB // 2


@functools.partial(
    pl.kernel,
    mesh=plsc.VectorSubcoreMesh(core_axis_name="c", subcore_axis_name="s"),
    out_type=jax.ShapeDtypeStruct((_FIELDS, _DIM2, _BATCH), jnp.float32),
    scratch_types=[
        pltpu.VMEM((_NCH, _CH), jnp.int32),
        pltpu.VMEM((2 * _SUB, _DIM2), jnp.float32),
        pltpu.VMEM((_FIELDS, _DIM2, _SUBB), jnp.float32),
        pltpu.VMEM((_FIELDS, _DIM2, _SUBB), jnp.float32),
        pltpu.SemaphoreType.DMA,
    ],
    compiler_params=pltpu.CompilerParams(
        use_tc_tiling_on_sc=False, needs_layout_passes=False),
)
def _gather_t(x_hbm, table_hbm, out_hbm, idx_v, rbuf, tb0, tb1, gsem):
    wid = lax.axis_index("s") * _NC + lax.axis_index("c")
    pltpu.sync_copy(x_hbm.at[wid], idx_v)
    iota = lax.iota(jnp.int32, 16)
    row_base = iota * _FIELDS

    def fire(s, half):
        return [
            pltpu.async_copy(
                table_hbm.at[idx_v.at[s * _CPS + k]],
                rbuf.at[pl.ds(half * _SUB + k * _CH, _CH)],
                gsem)
            for k in range(_CPS)
        ]

    def transpose(half, tb):
        src_off = half * _SUB

        @pl.loop(0, _DIM2)
        def _c(c):
            colv = jnp.full((16,), 0, jnp.int32) + c

            @pl.loop(0, _FIELDS, unroll=13)
            def _f(f):
                for h in (0, 1):
                    rowv = row_base + (src_off + 416 * h + f)
                    v = plsc.load_gather(rbuf, [rowv, colv])
                    tb[f, c, pl.ds(16 * h, 16)] = v

    @pl.loop(0, _NP)
    def _pairs(p):
        s0 = 2 * p
        h0 = fire(s0, 0)
        h1 = fire(s0 + 1, 1)
        for h in h0:
            h.wait()
        transpose(0, tb0)
        b0 = wid * _BW + s0 * _SUBB
        pltpu.sync_copy(tb0, out_hbm.at[:, :, pl.ds(b0, _SUBB)])
        for h in h1:
            h.wait()
        transpose(1, tb1)
        pltpu.sync_copy(tb1, out_hbm.at[:, :, pl.ds(b0 + _SUBB, _SUBB)])


def kernel(x, table):
    xw = x.reshape(_NW, _NCH, _CH)
    out_t = _gather_t(xw, table)
    return out_t.transpose(2, 0, 1)


# speedup vs baseline: 1.1718x; 1.1718x over previous
import functools

import jax
import jax.numpy as jnp
from jax import lax
from jax.experimental import pallas as pl
from jax.experimental.pallas import tpu as pltpu
from jax.experimental.pallas import tpu_sc as plsc

_DIM2 = 32
_BATCH = 16384
_FIELDS = 26
_B = _BATCH * _FIELDS

_info = plsc.get_sparse_core_info()
_NC = _info.num_cores
_NS = _info.num_subcores
_NW = _NC * _NS
_B_PER_W = _B // _NW
_CHUNK = 128
_N_CHUNKS = _B_PER_W // _CHUNK
_K = 13
_GROUP = _K * _CHUNK
_NG = _N_CHUNKS // _K


@functools.partial(
    pl.kernel,
    mesh=plsc.VectorSubcoreMesh(core_axis_name="c", subcore_axis_name="s"),
    out_type=jax.ShapeDtypeStruct((_NW, _B_PER_W, _DIM2), jnp.float32),
    scratch_types=[
        pltpu.VMEM((_N_CHUNKS, _CHUNK), jnp.int32),
        pltpu.VMEM((_GROUP, _DIM2), jnp.float32),
        pltpu.VMEM((_GROUP, _DIM2), jnp.float32),
        pltpu.SemaphoreType.DMA,
        pltpu.SemaphoreType.DMA,
        pltpu.SemaphoreType.DMA,
        pltpu.SemaphoreType.DMA,
    ],
    compiler_params=pltpu.CompilerParams(use_tc_tiling_on_sc=False),
)
def _gather_sc(x_hbm, table_hbm, out_hbm, idx_v, buf0, buf1, g0, g1, w0, w1):
    wid = lax.axis_index("s") * _NC + lax.axis_index("c")
    pltpu.sync_copy(x_hbm.at[wid], idx_v)
    bufs, gsems, wsems = (buf0, buf1), (g0, g1), (w0, w1)

    def wait_write(b):
        pltpu.make_async_copy(
            bufs[b], out_hbm.at[wid, pl.ds(0, _GROUP)], wsems[b]).wait()

    def do_group(gg, b, first):
        if not first:
            wait_write(b)
        handles = [
            pltpu.async_copy(
                table_hbm.at[idx_v.at[gg * _K + k]],
                bufs[b].at[pl.ds(k * _CHUNK, _CHUNK)],
                gsems[b])
            for k in range(_K)
        ]
        for h in handles:
            h.wait()
        pltpu.async_copy(
            bufs[b], out_hbm.at[wid, pl.ds(gg * _GROUP, _GROUP)], wsems[b])

    do_group(0, 0, True)
    do_group(1, 1, True)

    @pl.loop(0, _NG - 2, step=2)
    def _groups(g):
        do_group(g + 2, 0, False)
        do_group(g + 3, 1, False)

    wait_write(0)
    wait_write(1)


def kernel(x, table):
    xw = x.reshape(_NW, _N_CHUNKS, _CHUNK)
    out = _gather_sc(xw, table)
    return out.reshape(_BATCH, _FIELDS, _DIM2)
